# Initial kernel scaffold; baseline (speedup 1.0000x reference)
import jax
import jax.numpy as jnp
from jax.experimental import pallas as pl


def kernel(rnode_features, pnode_features, edge_features, senders, receivers, tau, params):
    raise NotImplementedError("write your pallas kernel here")


# speedup vs baseline: 3.5271x; 3.5271x over previous
import functools

import jax
import jax.numpy as jnp
from jax import lax
from jax.experimental import pallas as pl
from jax.experimental.pallas import tpu as pltpu
from jax.experimental.pallas import tpu_sc as plsc

NR = 2500
NP_ = 10000
E = 160000
D = 128
DE = 16
OUT = 4
CH = 16

NC = 2
NS = 16
NT = NC * NS
CHUNK = 128
NCH = 40
TILE_E = NCH * CHUNK
EP = NT * TILE_E
HALF = 5056
JUNK = 64
ACCR = HALF + JUNK
TILE_ES = EP // NS
SNCH = TILE_ES // CHUNK

def _gather_body(rn_hbm, pn_hbm, snd_hbm, rcv_hbm, rs_hbm, ps_hbm,
                 sidx, ridx, rbuf, pbuf, sem_r, sem_p):
    c = lax.axis_index("c")
    s = lax.axis_index("s")
    wid = s * NC + c
    base = wid * TILE_E
    pltpu.sync_copy(snd_hbm.at[wid], sidx)
    pltpu.sync_copy(rcv_hbm.at[wid], ridx)

    def body(j, _):
        cp_r = pltpu.async_copy(rn_hbm.at[sidx.at[j]], rbuf, sem_r)
        cp_p = pltpu.async_copy(pn_hbm.at[ridx.at[j]], pbuf, sem_p)
        cp_r.wait()
        cp_p.wait()
        off = base + j * CHUNK
        pltpu.sync_copy(rbuf, rs_hbm.at[pl.ds(off, CHUNK)])
        pltpu.sync_copy(pbuf, ps_hbm.at[pl.ds(off, CHUNK)])
        return 0

    lax.fori_loop(0, NCH, body, 0)


@functools.lru_cache(maxsize=None)
def _sc_kernels():
    mesh = plsc.VectorSubcoreMesh(
        core_axis_name="c", subcore_axis_name="s",
        num_cores=NC, num_subcores=NS)
    gather = pl.kernel(
        _gather_body,
        out_type=[
            jax.ShapeDtypeStruct((EP, D), jnp.float32),
            jax.ShapeDtypeStruct((EP, D), jnp.float32),
        ],
        mesh=mesh,
        scratch_types=[
            pltpu.VMEM((NCH, CHUNK), jnp.int32),
            pltpu.VMEM((NCH, CHUNK), jnp.int32),
            pltpu.VMEM((CHUNK, D), jnp.float32),
            pltpu.VMEM((CHUNK, D), jnp.float32),
            pltpu.SemaphoreType.DMA,
            pltpu.SemaphoreType.DMA,
        ],
    )
    scatter = pl.kernel(
        _scatter_body,
        out_type=[
            jax.ShapeDtypeStruct((NC, HALF, D), jnp.float32),
            jax.ShapeDtypeStruct((NC, HALF, D), jnp.float32),
        ],
        mesh=mesh,
        scratch_types=[
            pltpu.VMEM((SNCH, CHUNK), jnp.int32),
            pltpu.VMEM((CHUNK, D), jnp.float32),
            pltpu.VMEM((CHUNK, D), jnp.float32),
            pltpu.VMEM_SHARED((ACCR, D), jnp.float32),
            pltpu.VMEM_SHARED((ACCR, D), jnp.float32),
        ],
    )
    return gather, scatter


def _scatter_body(e_hbm, rcv_hbm, zsum_hbm, ones_hbm,
                  psum_hbm, pcnt_hbm, idx, ebuf, ones_v, acc, cacc):
    c = lax.axis_index("c")
    s = lax.axis_index("s")
    base = s * TILE_ES
    lo = c * HALF
    pltpu.sync_copy(zsum_hbm, ebuf)
    nblk = ACCR // CHUNK
    nmax = (nblk + NS - 1) // NS

    def zinit(k, _):
        blk = s + k * NS

        @pl.when(blk < nblk)
        def _():
            pltpu.sync_copy(ebuf, acc.at[pl.ds(blk * CHUNK, CHUNK)])
            pltpu.sync_copy(ebuf, cacc.at[pl.ds(blk * CHUNK, CHUNK)])

        return 0

    lax.fori_loop(0, nmax, zinit, 0)
    pltpu.sync_copy(ones_hbm, ones_v)
    pltpu.sync_copy(rcv_hbm.at[s], idx)
    iota = lax.iota(jnp.int32, 16)

    def locz(j, _):
        for k in range(CHUNK // 16):
            g = idx[j, pl.ds(k * 16, 16)]
            t = g - lo
            valid = (t >= 0) & (t < HALF)
            jv = HALF + iota + 16 * (k % (JUNK // 16))
            idx[j, pl.ds(k * 16, 16)] = jnp.where(valid, t, jv)
        return 0

    lax.fori_loop(0, SNCH, locz, 0)
    plsc.subcore_barrier()

    def body(j, _):
        pltpu.sync_copy(e_hbm.at[pl.ds(base + j * CHUNK, CHUNK)], ebuf)
        pltpu.sync_copy(ebuf, acc.at[idx.at[j]], add=True)
        pltpu.sync_copy(ones_v, cacc.at[idx.at[j]], add=True)
        return 0

    lax.fori_loop(0, SNCH, body, 0)
    plsc.subcore_barrier()
    wb = HALF // 64
    wmax = (wb + NS - 1) // NS

    def wback(k, _):
        blk = s + k * NS

        @pl.when(blk < wb)
        def _():
            r0 = blk * 64
            pltpu.sync_copy(acc.at[pl.ds(r0, 64)], ebuf.at[pl.ds(0, 64)])
            pltpu.sync_copy(ebuf.at[pl.ds(0, 64)],
                            psum_hbm.at[c, pl.ds(r0, 64)])
            pltpu.sync_copy(cacc.at[pl.ds(r0, 64)], ebuf.at[pl.ds(64, 64)])
            pltpu.sync_copy(ebuf.at[pl.ds(64, 64)],
                            pcnt_hbm.at[c, pl.ds(r0, 64)])

        return 0

    lax.fori_loop(0, wmax, wback, 0)


def _swish(x):
    return x * jax.nn.sigmoid(x)


def _cn_vectors(tau, cw, cb, sw, sb, ow, ob):
    h = _swish(tau * cw + cb)
    scale = jnp.dot(h, sw, preferred_element_type=jnp.float32) + sb
    shift = jnp.dot(h, ow, preferred_element_type=jnp.float32) + ob
    return scale, shift


def _layernorm(x):
    m = jnp.mean(x, axis=-1, keepdims=True)
    xc = x - m
    v = jnp.mean(xc * xc, axis=-1, keepdims=True)
    return xc * lax.rsqrt(v + 1e-5)


def _edge_body(tau_ref, ef_ref, rs_ref, ps_ref,
               W_em1, b_em1, W_em2, b_em2,
               em_cw, em_cb, em_sw, em_sb, em_ow, em_ob,
               W_pe1, b_pe1, W_pe2, b_pe2,
               pe_cw, pe_cb, pe_sw, pe_sb, pe_ow, pe_ob,
               e_ref):
    tau = tau_ref[0, 0]
    sc_em, sh_em = _cn_vectors(tau, em_cw[...], em_cb[...], em_sw[...],
                               em_sb[...], em_ow[...], em_ob[...])
    sc_pe, sh_pe = _cn_vectors(tau, pe_cw[...], pe_cb[...], pe_sw[...],
                               pe_sb[...], pe_ow[...], pe_ob[...])
    x = ef_ref[...]
    h = _swish(jnp.dot(x, W_em1[...], preferred_element_type=jnp.float32)
               + b_em1[...])
    e0 = jnp.dot(h, W_em2[...], preferred_element_type=jnp.float32) + b_em2[...]
    e0 = _layernorm(e0) * (1.0 + sc_em) + sh_em
    m = (jnp.dot(e0, W_pe1[...][:D], preferred_element_type=jnp.float32)
         + jnp.dot(rs_ref[...], W_pe1[...][D:2 * D],
                   preferred_element_type=jnp.float32)
         + jnp.dot(ps_ref[...], W_pe1[...][2 * D:],
                   preferred_element_type=jnp.float32)
         + b_pe1[...])
    m = jnp.dot(_swish(m), W_pe2[...], preferred_element_type=jnp.float32) \
        + b_pe2[...]
    m = _layernorm(m) * (1.0 + sc_pe) + sh_pe
    e_ref[...] = e0 + m


def _node_body(tau_ref, pn_ref, psum_ref, pcnt_ref,
               W_pn1, b_pn1, W_pn2, b_pn2,
               pn_cw, pn_cb, pn_sw, pn_sb, pn_ow, pn_ob,
               W_d1, b_d1, W_d2, b_d2,
               out_ref):
    tau = tau_ref[0, 0]
    sc_pn, sh_pn = _cn_vectors(tau, pn_cw[...], pn_cb[...], pn_sw[...],
                               pn_sb[...], pn_ow[...], pn_ob[...])
    pn = pn_ref[...]
    s = psum_ref[0]
    cnt = pcnt_ref[0, :, 0:1]
    agg = s / jnp.maximum(cnt, 1.0)
    h = _swish(jnp.dot(pn, W_pn1[...][:D], preferred_element_type=jnp.float32)
               + jnp.dot(agg, W_pn1[...][D:],
                         preferred_element_type=jnp.float32)
               + b_pn1[...])
    h = jnp.dot(h, W_pn2[...], preferred_element_type=jnp.float32) + b_pn2[...]
    h = _layernorm(h) * (1.0 + sc_pn) + sh_pn
    pn = pn + h
    d = _swish(jnp.dot(pn, W_d1[...], preferred_element_type=jnp.float32)
               + b_d1[...])
    out_ref[...] = jnp.dot(d, W_d2[...], preferred_element_type=jnp.float32) \
        + b_d2[...]


def _full(shape):
    nd = len(shape)
    return pl.BlockSpec(shape, lambda i: (0,) * nd)


def kernel(rnode_features, pnode_features, edge_features, senders, receivers,
           tau, params):
    rn = rnode_features[0]
    pn = pnode_features[0]
    ef = edge_features[0]
    snd = senders[0]
    rcv = receivers[0]
    p = params

    pad = EP - E
    ef_p = jnp.concatenate([ef, jnp.zeros((pad, DE), jnp.float32)], axis=0)
    snd3 = jnp.concatenate([snd, jnp.zeros((pad,), jnp.int32)]) \
        .reshape(NT, NCH, CHUNK)
    rcv3 = jnp.concatenate([rcv, jnp.full((pad,), NP_, jnp.int32)]) \
        .reshape(NT, NCH, CHUNK)
    tau_arr = jnp.reshape(tau, (1, 1)).astype(jnp.float32)

    gather_sc, scatter_sc = _sc_kernels()
    rs, ps = gather_sc(rn, pn, snd3, rcv3)

    BLK = 1024
    r1 = lambda a: a.reshape(1, -1)
    edge_w = [p['W_em1'], r1(p['b_em1']), p['W_em2'], r1(p['b_em2']),
              p['em_cw'], r1(p['em_cb']), p['em_sw'], r1(p['em_sb']),
              p['em_ow'], r1(p['em_ob']),
              p['W_pe1'], r1(p['b_pe1']), p['W_pe2'], r1(p['b_pe2']),
              p['pe_cw'], r1(p['pe_cb']), p['pe_sw'], r1(p['pe_sb']),
              p['pe_ow'], r1(p['pe_ob'])]
    e = pl.pallas_call(
        _edge_body,
        grid=(EP // BLK,),
        in_specs=[
            pl.BlockSpec((1, 1), lambda i: (0, 0), memory_space=pltpu.SMEM),
            pl.BlockSpec((BLK, DE), lambda i: (i, 0)),
            pl.BlockSpec((BLK, D), lambda i: (i, 0)),
            pl.BlockSpec((BLK, D), lambda i: (i, 0)),
        ] + [_full(w.shape) for w in edge_w],
        out_specs=pl.BlockSpec((BLK, D), lambda i: (i, 0)),
        out_shape=jax.ShapeDtypeStruct((EP, D), jnp.float32),
    )(tau_arr, ef_p, rs, ps, *edge_w)

    zsum = jnp.zeros((CHUNK, D), jnp.float32)
    ones_c = jnp.ones((CHUNK, D), jnp.float32)
    rcv3s = rcv3.reshape(NS, SNCH, CHUNK)
    psum, pcnt = scatter_sc(e, rcv3s, zsum, ones_c)

    BLKN = HALF // 2
    node_w = [p['W_pn1'], r1(p['b_pn1']), p['W_pn2'], r1(p['b_pn2']),
              p['pn_cw'], r1(p['pn_cb']), p['pn_sw'], r1(p['pn_sb']),
              p['pn_ow'], r1(p['pn_ob']),
              p['W_d1'], r1(p['b_d1']), p['W_d2'], r1(p['b_d2'])]
    out = pl.pallas_call(
        _node_body,
        grid=(2 * NC,),
        in_specs=[
            pl.BlockSpec((1, 1), lambda i: (0, 0), memory_space=pltpu.SMEM),
            pl.BlockSpec((BLKN, D), lambda i: (i, 0)),
            pl.BlockSpec((1, BLKN, D), lambda i: (i // 2, i % 2, 0)),
            pl.BlockSpec((1, BLKN, D), lambda i: (i // 2, i % 2, 0)),
        ] + [_full(w.shape) for w in node_w],
        out_specs=pl.BlockSpec((BLKN, OUT), lambda i: (i, 0)),
        out_shape=jax.ShapeDtypeStruct((NP_, OUT), jnp.float32),
    )(tau_arr, pn, psum, pcnt, *node_w)

    return out.reshape(1, NP_, OUT)


# speedup vs baseline: 3.5505x; 1.0066x over previous
import functools

import jax
import jax.numpy as jnp
from jax import lax
from jax.experimental import pallas as pl
from jax.experimental.pallas import tpu as pltpu
from jax.experimental.pallas import tpu_sc as plsc

NR = 2500
NP_ = 10000
E = 160000
D = 128
DE = 16
OUT = 4
CH = 16

NC = 2
NS = 16
NT = NC * NS
CHUNK = 128
NCH = 40
TILE_E = NCH * CHUNK
EP = NT * TILE_E
HALF = 5056
JUNK = 64
ACCR = HALF + JUNK
TILE_ES = EP // NS
SNCH = TILE_ES // CHUNK

def _gather_body(rn_hbm, pn_hbm, snd_hbm, rcv_hbm, rs_hbm, ps_hbm,
                 sidx, ridx, rb0, pb0, rb1, pb1, sg0, sg1, sw0, sw1):
    c = lax.axis_index("c")
    s = lax.axis_index("s")
    wid = s * NC + c
    base = wid * TILE_E
    pltpu.sync_copy(snd_hbm.at[wid], sidx)
    pltpu.sync_copy(rcv_hbm.at[wid], ridx)

    def wofs(j):
        return base + j * CHUNK

    pltpu.async_copy(rn_hbm.at[sidx.at[0]], rb0, sg0)
    pltpu.async_copy(pn_hbm.at[ridx.at[0]], pb0, sg0)

    def pair(k, _):
        j0 = 2 * k
        j1 = j0 + 1
        pltpu.make_async_copy(rn_hbm.at[sidx.at[j0]], rb0, sg0).wait()
        pltpu.make_async_copy(pn_hbm.at[ridx.at[j0]], pb0, sg0).wait()

        @pl.when(k > 0)
        def _():
            pltpu.make_async_copy(rb1, rs_hbm.at[pl.ds(wofs(j0 - 1), CHUNK)],
                                  sw1).wait()
            pltpu.make_async_copy(pb1, ps_hbm.at[pl.ds(wofs(j0 - 1), CHUNK)],
                                  sw1).wait()

        pltpu.async_copy(rn_hbm.at[sidx.at[j1]], rb1, sg1)
        pltpu.async_copy(pn_hbm.at[ridx.at[j1]], pb1, sg1)
        pltpu.async_copy(rb0, rs_hbm.at[pl.ds(wofs(j0), CHUNK)], sw0)
        pltpu.async_copy(pb0, ps_hbm.at[pl.ds(wofs(j0), CHUNK)], sw0)

        pltpu.make_async_copy(rn_hbm.at[sidx.at[j1]], rb1, sg1).wait()
        pltpu.make_async_copy(pn_hbm.at[ridx.at[j1]], pb1, sg1).wait()
        pltpu.make_async_copy(rb0, rs_hbm.at[pl.ds(wofs(j0), CHUNK)],
                              sw0).wait()
        pltpu.make_async_copy(pb0, ps_hbm.at[pl.ds(wofs(j0), CHUNK)],
                              sw0).wait()

        @pl.when(k < NCH // 2 - 1)
        def _():
            pltpu.async_copy(rn_hbm.at[sidx.at[j0 + 2]], rb0, sg0)
            pltpu.async_copy(pn_hbm.at[ridx.at[j0 + 2]], pb0, sg0)

        pltpu.async_copy(rb1, rs_hbm.at[pl.ds(wofs(j1), CHUNK)], sw1)
        pltpu.async_copy(pb1, ps_hbm.at[pl.ds(wofs(j1), CHUNK)], sw1)
        return 0

    lax.fori_loop(0, NCH // 2, pair, 0)
    last = NCH - 1
    pltpu.make_async_copy(rb1, rs_hbm.at[pl.ds(wofs(last), CHUNK)], sw1).wait()
    pltpu.make_async_copy(pb1, ps_hbm.at[pl.ds(wofs(last), CHUNK)], sw1).wait()


@functools.lru_cache(maxsize=None)
def _sc_kernels():
    mesh = plsc.VectorSubcoreMesh(
        core_axis_name="c", subcore_axis_name="s",
        num_cores=NC, num_subcores=NS)
    gather = pl.kernel(
        _gather_body,
        out_type=[
            jax.ShapeDtypeStruct((EP, D), jnp.float32),
            jax.ShapeDtypeStruct((EP, D), jnp.float32),
        ],
        mesh=mesh,
        scratch_types=[
            pltpu.VMEM((NCH, CHUNK), jnp.int32),
            pltpu.VMEM((NCH, CHUNK), jnp.int32),
            pltpu.VMEM((CHUNK, D), jnp.float32),
            pltpu.VMEM((CHUNK, D), jnp.float32),
            pltpu.VMEM((CHUNK, D), jnp.float32),
            pltpu.VMEM((CHUNK, D), jnp.float32),
            pltpu.SemaphoreType.DMA,
            pltpu.SemaphoreType.DMA,
            pltpu.SemaphoreType.DMA,
            pltpu.SemaphoreType.DMA,
        ],
    )
    scatter = pl.kernel(
        _scatter_body,
        out_type=[
            jax.ShapeDtypeStruct((NC, HALF, D), jnp.float32),
            jax.ShapeDtypeStruct((NC, HALF, D), jnp.float32),
        ],
        mesh=mesh,
        scratch_types=[
            pltpu.VMEM((SNCH, CHUNK), jnp.int32),
            pltpu.VMEM((CHUNK, D), jnp.float32),
            pltpu.VMEM((CHUNK, D), jnp.float32),
            pltpu.VMEM_SHARED((ACCR, D), jnp.float32),
            pltpu.VMEM_SHARED((ACCR, D), jnp.float32),
        ],
    )
    return gather, scatter


def _scatter_body(e_hbm, rcv_hbm, zsum_hbm, ones_hbm,
                  psum_hbm, pcnt_hbm, idx, ebuf, ones_v, acc, cacc):
    c = lax.axis_index("c")
    s = lax.axis_index("s")
    base = s * TILE_ES
    lo = c * HALF
    pltpu.sync_copy(zsum_hbm, ebuf)
    nblk = ACCR // CHUNK
    nmax = (nblk + NS - 1) // NS

    def zinit(k, _):
        blk = s + k * NS

        @pl.when(blk < nblk)
        def _():
            pltpu.sync_copy(ebuf, acc.at[pl.ds(blk * CHUNK, CHUNK)])
            pltpu.sync_copy(ebuf, cacc.at[pl.ds(blk * CHUNK, CHUNK)])

        return 0

    lax.fori_loop(0, nmax, zinit, 0)
    pltpu.sync_copy(ones_hbm, ones_v)
    pltpu.sync_copy(rcv_hbm.at[s], idx)
    iota = lax.iota(jnp.int32, 16)

    def locz(j, _):
        for k in range(CHUNK // 16):
            g = idx[j, pl.ds(k * 16, 16)]
            t = g - lo
            valid = (t >= 0) & (t < HALF)
            jv = HALF + iota + 16 * (k % (JUNK // 16))
            idx[j, pl.ds(k * 16, 16)] = jnp.where(valid, t, jv)
        return 0

    lax.fori_loop(0, SNCH, locz, 0)
    plsc.subcore_barrier()

    def body(j, _):
        pltpu.sync_copy(e_hbm.at[pl.ds(base + j * CHUNK, CHUNK)], ebuf)
        pltpu.sync_copy(ebuf, acc.at[idx.at[j]], add=True)
        pltpu.sync_copy(ones_v, cacc.at[idx.at[j]], add=True)
        return 0

    lax.fori_loop(0, SNCH, body, 0)
    plsc.subcore_barrier()
    wb = HALF // 64
    wmax = (wb + NS - 1) // NS

    def wback(k, _):
        blk = s + k * NS

        @pl.when(blk < wb)
        def _():
            r0 = blk * 64
            pltpu.sync_copy(acc.at[pl.ds(r0, 64)], ebuf.at[pl.ds(0, 64)])
            pltpu.sync_copy(ebuf.at[pl.ds(0, 64)],
                            psum_hbm.at[c, pl.ds(r0, 64)])
            pltpu.sync_copy(cacc.at[pl.ds(r0, 64)], ebuf.at[pl.ds(64, 64)])
            pltpu.sync_copy(ebuf.at[pl.ds(64, 64)],
                            pcnt_hbm.at[c, pl.ds(r0, 64)])

        return 0

    lax.fori_loop(0, wmax, wback, 0)


def _swish(x):
    return x * jax.nn.sigmoid(x)


def _cn_vectors(tau, cw, cb, sw, sb, ow, ob):
    h = _swish(tau * cw + cb)
    scale = jnp.dot(h, sw, preferred_element_type=jnp.float32) + sb
    shift = jnp.dot(h, ow, preferred_element_type=jnp.float32) + ob
    return scale, shift


def _layernorm(x):
    m = jnp.mean(x, axis=-1, keepdims=True)
    xc = x - m
    v = jnp.mean(xc * xc, axis=-1, keepdims=True)
    return xc * lax.rsqrt(v + 1e-5)


def _edge_body(tau_ref, ef_ref, rs_ref, ps_ref,
               W_em1, b_em1, W_em2, b_em2,
               em_cw, em_cb, em_sw, em_sb, em_ow, em_ob,
               W_pe1, b_pe1, W_pe2, b_pe2,
               pe_cw, pe_cb, pe_sw, pe_sb, pe_ow, pe_ob,
               e_ref):
    tau = tau_ref[0, 0]
    sc_em, sh_em = _cn_vectors(tau, em_cw[...], em_cb[...], em_sw[...],
                               em_sb[...], em_ow[...], em_ob[...])
    sc_pe, sh_pe = _cn_vectors(tau, pe_cw[...], pe_cb[...], pe_sw[...],
                               pe_sb[...], pe_ow[...], pe_ob[...])
    x = ef_ref[...]
    h = _swish(jnp.dot(x, W_em1[...], preferred_element_type=jnp.float32)
               + b_em1[...])
    e0 = jnp.dot(h, W_em2[...], preferred_element_type=jnp.float32) + b_em2[...]
    e0 = _layernorm(e0) * (1.0 + sc_em) + sh_em
    m = (jnp.dot(e0, W_pe1[...][:D], preferred_element_type=jnp.float32)
         + jnp.dot(rs_ref[...], W_pe1[...][D:2 * D],
                   preferred_element_type=jnp.float32)
         + jnp.dot(ps_ref[...], W_pe1[...][2 * D:],
                   preferred_element_type=jnp.float32)
         + b_pe1[...])
    m = jnp.dot(_swish(m), W_pe2[...], preferred_element_type=jnp.float32) \
        + b_pe2[...]
    m = _layernorm(m) * (1.0 + sc_pe) + sh_pe
    e_ref[...] = e0 + m


def _node_body(tau_ref, pn_ref, psum_ref, pcnt_ref,
               W_pn1, b_pn1, W_pn2, b_pn2,
               pn_cw, pn_cb, pn_sw, pn_sb, pn_ow, pn_ob,
               W_d1, b_d1, W_d2, b_d2,
               out_ref):
    tau = tau_ref[0, 0]
    sc_pn, sh_pn = _cn_vectors(tau, pn_cw[...], pn_cb[...], pn_sw[...],
                               pn_sb[...], pn_ow[...], pn_ob[...])
    pn = pn_ref[...]
    s = psum_ref[0]
    cnt = pcnt_ref[0, :, 0:1]
    agg = s / jnp.maximum(cnt, 1.0)
    h = _swish(jnp.dot(pn, W_pn1[...][:D], preferred_element_type=jnp.float32)
               + jnp.dot(agg, W_pn1[...][D:],
                         preferred_element_type=jnp.float32)
               + b_pn1[...])
    h = jnp.dot(h, W_pn2[...], preferred_element_type=jnp.float32) + b_pn2[...]
    h = _layernorm(h) * (1.0 + sc_pn) + sh_pn
    pn = pn + h
    d = _swish(jnp.dot(pn, W_d1[...], preferred_element_type=jnp.float32)
               + b_d1[...])
    out_ref[...] = jnp.dot(d, W_d2[...], preferred_element_type=jnp.float32) \
        + b_d2[...]


def _full(shape):
    nd = len(shape)
    return pl.BlockSpec(shape, lambda i: (0,) * nd)


def kernel(rnode_features, pnode_features, edge_features, senders, receivers,
           tau, params):
    rn = rnode_features[0]
    pn = pnode_features[0]
    ef = edge_features[0]
    snd = senders[0]
    rcv = receivers[0]
    p = params

    pad = EP - E
    ef_p = jnp.concatenate([ef, jnp.zeros((pad, DE), jnp.float32)], axis=0)
    snd3 = jnp.concatenate([snd, jnp.zeros((pad,), jnp.int32)]) \
        .reshape(NT, NCH, CHUNK)
    rcv3 = jnp.concatenate([rcv, jnp.full((pad,), NP_, jnp.int32)]) \
        .reshape(NT, NCH, CHUNK)
    tau_arr = jnp.reshape(tau, (1, 1)).astype(jnp.float32)

    gather_sc, scatter_sc = _sc_kernels()
    rs, ps = gather_sc(rn, pn, snd3, rcv3)

    BLK = 1024
    r1 = lambda a: a.reshape(1, -1)
    edge_w = [p['W_em1'], r1(p['b_em1']), p['W_em2'], r1(p['b_em2']),
              p['em_cw'], r1(p['em_cb']), p['em_sw'], r1(p['em_sb']),
              p['em_ow'], r1(p['em_ob']),
              p['W_pe1'], r1(p['b_pe1']), p['W_pe2'], r1(p['b_pe2']),
              p['pe_cw'], r1(p['pe_cb']), p['pe_sw'], r1(p['pe_sb']),
              p['pe_ow'], r1(p['pe_ob'])]
    e = pl.pallas_call(
        _edge_body,
        grid=(EP // BLK,),
        in_specs=[
            pl.BlockSpec((1, 1), lambda i: (0, 0), memory_space=pltpu.SMEM),
            pl.BlockSpec((BLK, DE), lambda i: (i, 0)),
            pl.BlockSpec((BLK, D), lambda i: (i, 0)),
            pl.BlockSpec((BLK, D), lambda i: (i, 0)),
        ] + [_full(w.shape) for w in edge_w],
        out_specs=pl.BlockSpec((BLK, D), lambda i: (i, 0)),
        out_shape=jax.ShapeDtypeStruct((EP, D), jnp.float32),
    )(tau_arr, ef_p, rs, ps, *edge_w)

    zsum = jnp.zeros((CHUNK, D), jnp.float32)
    ones_c = jnp.ones((CHUNK, D), jnp.float32)
    rcv3s = rcv3.reshape(NS, SNCH, CHUNK)
    psum, pcnt = scatter_sc(e, rcv3s, zsum, ones_c)

    BLKN = HALF // 2
    node_w = [p['W_pn1'], r1(p['b_pn1']), p['W_pn2'], r1(p['b_pn2']),
              p['pn_cw'], r1(p['pn_cb']), p['pn_sw'], r1(p['pn_sb']),
              p['pn_ow'], r1(p['pn_ob']),
              p['W_d1'], r1(p['b_d1']), p['W_d2'], r1(p['b_d2'])]
    out = pl.pallas_call(
        _node_body,
        grid=(2 * NC,),
        in_specs=[
            pl.BlockSpec((1, 1), lambda i: (0, 0), memory_space=pltpu.SMEM),
            pl.BlockSpec((BLKN, D), lambda i: (i, 0)),
            pl.BlockSpec((1, BLKN, D), lambda i: (i // 2, i % 2, 0)),
            pl.BlockSpec((1, BLKN, D), lambda i: (i // 2, i % 2, 0)),
        ] + [_full(w.shape) for w in node_w],
        out_specs=pl.BlockSpec((BLKN, OUT), lambda i: (i, 0)),
        out_shape=jax.ShapeDtypeStruct((NP_, OUT), jnp.float32),
    )(tau_arr, pn, psum, pcnt, *node_w)

    return out.reshape(1, NP_, OUT)


# speedup vs baseline: 4.3170x; 1.2159x over previous
import functools

import jax
import jax.numpy as jnp
from jax import lax
from jax.experimental import pallas as pl
from jax.experimental.pallas import tpu as pltpu
from jax.experimental.pallas import tpu_sc as plsc

NR = 2500
NP_ = 10000
E = 160000
D = 128
DE = 16
OUT = 4
CH = 16

NC = 2
NS = 16
NT = NC * NS
CHUNK = 128
EP = 163840
NH = 2
EH = EP // NH
TILE_E = EH // NT
NCH = TILE_E // CHUNK
HALF = 5056
JUNK = 64
ACCR = HALF + JUNK
TILE_ES = EH // NS
SNCH = TILE_ES // CHUNK

def _gather_body(rn_hbm, pn_hbm, snd_hbm, rcv_hbm, rs_hbm, ps_hbm,
                 sidx, ridx, rb0, pb0, rb1, pb1, sg0, sg1, sw0, sw1):
    c = lax.axis_index("c")
    s = lax.axis_index("s")
    wid = s * NC + c
    base = wid * TILE_E
    pltpu.sync_copy(snd_hbm.at[wid], sidx)
    pltpu.sync_copy(rcv_hbm.at[wid], ridx)

    def wofs(j):
        return base + j * CHUNK

    pltpu.async_copy(rn_hbm.at[sidx.at[0]], rb0, sg0)
    pltpu.async_copy(pn_hbm.at[ridx.at[0]], pb0, sg0)

    def pair(k, _):
        j0 = 2 * k
        j1 = j0 + 1
        pltpu.make_async_copy(rn_hbm.at[sidx.at[j0]], rb0, sg0).wait()
        pltpu.make_async_copy(pn_hbm.at[ridx.at[j0]], pb0, sg0).wait()

        @pl.when(k > 0)
        def _():
            pltpu.make_async_copy(rb1, rs_hbm.at[pl.ds(wofs(j0 - 1), CHUNK)],
                                  sw1).wait()
            pltpu.make_async_copy(pb1, ps_hbm.at[pl.ds(wofs(j0 - 1), CHUNK)],
                                  sw1).wait()

        pltpu.async_copy(rn_hbm.at[sidx.at[j1]], rb1, sg1)
        pltpu.async_copy(pn_hbm.at[ridx.at[j1]], pb1, sg1)
        pltpu.async_copy(rb0, rs_hbm.at[pl.ds(wofs(j0), CHUNK)], sw0)
        pltpu.async_copy(pb0, ps_hbm.at[pl.ds(wofs(j0), CHUNK)], sw0)

        pltpu.make_async_copy(rn_hbm.at[sidx.at[j1]], rb1, sg1).wait()
        pltpu.make_async_copy(pn_hbm.at[ridx.at[j1]], pb1, sg1).wait()
        pltpu.make_async_copy(rb0, rs_hbm.at[pl.ds(wofs(j0), CHUNK)],
                              sw0).wait()
        pltpu.make_async_copy(pb0, ps_hbm.at[pl.ds(wofs(j0), CHUNK)],
                              sw0).wait()

        @pl.when(k < NCH // 2 - 1)
        def _():
            pltpu.async_copy(rn_hbm.at[sidx.at[j0 + 2]], rb0, sg0)
            pltpu.async_copy(pn_hbm.at[ridx.at[j0 + 2]], pb0, sg0)

        pltpu.async_copy(rb1, rs_hbm.at[pl.ds(wofs(j1), CHUNK)], sw1)
        pltpu.async_copy(pb1, ps_hbm.at[pl.ds(wofs(j1), CHUNK)], sw1)
        return 0

    lax.fori_loop(0, NCH // 2, pair, 0)
    last = NCH - 1
    pltpu.make_async_copy(rb1, rs_hbm.at[pl.ds(wofs(last), CHUNK)], sw1).wait()
    pltpu.make_async_copy(pb1, ps_hbm.at[pl.ds(wofs(last), CHUNK)], sw1).wait()


@functools.lru_cache(maxsize=None)
def _sc_kernels():
    mesh = plsc.VectorSubcoreMesh(
        core_axis_name="c", subcore_axis_name="s",
        num_cores=NC, num_subcores=NS)
    gather = pl.kernel(
        _gather_body,
        out_type=[
            jax.ShapeDtypeStruct((EH, D), jnp.float32),
            jax.ShapeDtypeStruct((EH, D), jnp.float32),
        ],
        mesh=mesh,
        scratch_types=[
            pltpu.VMEM((NCH, CHUNK), jnp.int32),
            pltpu.VMEM((NCH, CHUNK), jnp.int32),
            pltpu.VMEM((CHUNK, D), jnp.float32),
            pltpu.VMEM((CHUNK, D), jnp.float32),
            pltpu.VMEM((CHUNK, D), jnp.float32),
            pltpu.VMEM((CHUNK, D), jnp.float32),
            pltpu.SemaphoreType.DMA,
            pltpu.SemaphoreType.DMA,
            pltpu.SemaphoreType.DMA,
            pltpu.SemaphoreType.DMA,
        ],
    )
    scatter = pl.kernel(
        _scatter_body,
        out_type=[
            jax.ShapeDtypeStruct((NC, HALF, D), jnp.float32),
            jax.ShapeDtypeStruct((NC, HALF, D), jnp.float32),
        ],
        mesh=mesh,
        scratch_types=[
            pltpu.VMEM((SNCH, CHUNK), jnp.int32),
            pltpu.VMEM((CHUNK, D), jnp.float32),
            pltpu.VMEM((CHUNK, D), jnp.float32),
            pltpu.VMEM_SHARED((ACCR, D), jnp.float32),
            pltpu.VMEM_SHARED((ACCR, D), jnp.float32),
        ],
    )
    return gather, scatter


def _scatter_body(e_hbm, rcv_hbm, zsum_hbm, ones_hbm,
                  psum_hbm, pcnt_hbm, idx, ebuf, ones_v, acc, cacc):
    c = lax.axis_index("c")
    s = lax.axis_index("s")
    base = s * TILE_ES
    lo = c * HALF
    pltpu.sync_copy(zsum_hbm, ebuf)
    nblk = ACCR // CHUNK
    nmax = (nblk + NS - 1) // NS

    def zinit(k, _):
        blk = s + k * NS

        @pl.when(blk < nblk)
        def _():
            pltpu.sync_copy(ebuf, acc.at[pl.ds(blk * CHUNK, CHUNK)])
            pltpu.sync_copy(ebuf, cacc.at[pl.ds(blk * CHUNK, CHUNK)])

        return 0

    lax.fori_loop(0, nmax, zinit, 0)
    pltpu.sync_copy(ones_hbm, ones_v)
    pltpu.sync_copy(rcv_hbm.at[s], idx)
    iota = lax.iota(jnp.int32, 16)

    def locz(j, _):
        for k in range(CHUNK // 16):
            g = idx[j, pl.ds(k * 16, 16)]
            t = g - lo
            valid = (t >= 0) & (t < HALF)
            jv = HALF + iota + 16 * (k % (JUNK // 16))
            idx[j, pl.ds(k * 16, 16)] = jnp.where(valid, t, jv)
        return 0

    lax.fori_loop(0, SNCH, locz, 0)
    plsc.subcore_barrier()

    def body(j, _):
        pltpu.sync_copy(e_hbm.at[pl.ds(base + j * CHUNK, CHUNK)], ebuf)
        pltpu.sync_copy(ebuf, acc.at[idx.at[j]], add=True)
        pltpu.sync_copy(ones_v, cacc.at[idx.at[j]], add=True)
        return 0

    lax.fori_loop(0, SNCH, body, 0)
    plsc.subcore_barrier()
    wb = HALF // 64
    wmax = (wb + NS - 1) // NS

    def wback(k, _):
        blk = s + k * NS

        @pl.when(blk < wb)
        def _():
            r0 = blk * 64
            pltpu.sync_copy(acc.at[pl.ds(r0, 64)], ebuf.at[pl.ds(0, 64)])
            pltpu.sync_copy(ebuf.at[pl.ds(0, 64)],
                            psum_hbm.at[c, pl.ds(r0, 64)])
            pltpu.sync_copy(cacc.at[pl.ds(r0, 64)], ebuf.at[pl.ds(64, 64)])
            pltpu.sync_copy(ebuf.at[pl.ds(64, 64)],
                            pcnt_hbm.at[c, pl.ds(r0, 64)])

        return 0

    lax.fori_loop(0, wmax, wback, 0)


def _swish(x):
    return x * jax.nn.sigmoid(x)


def _cn_vectors(tau, cw, cb, sw, sb, ow, ob):
    h = _swish(tau * cw + cb)
    scale = jnp.dot(h, sw, preferred_element_type=jnp.float32) + sb
    shift = jnp.dot(h, ow, preferred_element_type=jnp.float32) + ob
    return scale, shift


def _layernorm(x):
    m = jnp.mean(x, axis=-1, keepdims=True)
    xc = x - m
    v = jnp.mean(xc * xc, axis=-1, keepdims=True)
    return xc * lax.rsqrt(v + 1e-5)


def _edge_body(tau_ref, ef_ref, rs_ref, ps_ref,
               W_em1, b_em1, W_em2, b_em2,
               em_cw, em_cb, em_sw, em_sb, em_ow, em_ob,
               W_pe1, b_pe1, W_pe2, b_pe2,
               pe_cw, pe_cb, pe_sw, pe_sb, pe_ow, pe_ob,
               e_ref):
    tau = tau_ref[0, 0]
    sc_em, sh_em = _cn_vectors(tau, em_cw[...], em_cb[...], em_sw[...],
                               em_sb[...], em_ow[...], em_ob[...])
    sc_pe, sh_pe = _cn_vectors(tau, pe_cw[...], pe_cb[...], pe_sw[...],
                               pe_sb[...], pe_ow[...], pe_ob[...])
    x = ef_ref[...]
    h = _swish(jnp.dot(x, W_em1[...], preferred_element_type=jnp.float32)
               + b_em1[...])
    e0 = jnp.dot(h.astype(jnp.bfloat16), W_em2[...],
                 preferred_element_type=jnp.float32) + b_em2[...]
    e0 = _layernorm(e0) * (1.0 + sc_em) + sh_em
    m = (jnp.dot(e0.astype(jnp.bfloat16), W_pe1[...][:D],
                 preferred_element_type=jnp.float32)
         + jnp.dot(rs_ref[...], W_pe1[...][D:2 * D],
                   preferred_element_type=jnp.float32)
         + jnp.dot(ps_ref[...], W_pe1[...][2 * D:],
                   preferred_element_type=jnp.float32)
         + b_pe1[...])
    m = jnp.dot(_swish(m).astype(jnp.bfloat16), W_pe2[...],
                preferred_element_type=jnp.float32) \
        + b_pe2[...]
    m = _layernorm(m) * (1.0 + sc_pe) + sh_pe
    e_ref[...] = e0 + m


def _node_body(tau_ref, pn_ref, psum0_ref, pcnt0_ref, psum1_ref, pcnt1_ref,
               W_pn1, b_pn1, W_pn2, b_pn2,
               pn_cw, pn_cb, pn_sw, pn_sb, pn_ow, pn_ob,
               W_d1, b_d1, W_d2, b_d2,
               out_ref):
    tau = tau_ref[0, 0]
    sc_pn, sh_pn = _cn_vectors(tau, pn_cw[...], pn_cb[...], pn_sw[...],
                               pn_sb[...], pn_ow[...], pn_ob[...])
    pn = pn_ref[...]
    s = psum0_ref[0] + psum1_ref[0]
    cnt = pcnt0_ref[0, :, 0:1] + pcnt1_ref[0, :, 0:1]
    agg = s / jnp.maximum(cnt, 1.0)
    h = _swish(jnp.dot(pn, W_pn1[...][:D], preferred_element_type=jnp.float32)
               + jnp.dot(agg, W_pn1[...][D:],
                         preferred_element_type=jnp.float32)
               + b_pn1[...])
    h = jnp.dot(h, W_pn2[...], preferred_element_type=jnp.float32) + b_pn2[...]
    h = _layernorm(h) * (1.0 + sc_pn) + sh_pn
    pn = pn + h
    d = _swish(jnp.dot(pn, W_d1[...], preferred_element_type=jnp.float32)
               + b_d1[...])
    out_ref[...] = jnp.dot(d, W_d2[...], preferred_element_type=jnp.float32) \
        + b_d2[...]


def _full(shape):
    nd = len(shape)
    return pl.BlockSpec(shape, lambda i: (0,) * nd)


def kernel(rnode_features, pnode_features, edge_features, senders, receivers,
           tau, params):
    rn = rnode_features[0]
    pn = pnode_features[0]
    ef = edge_features[0]
    snd = senders[0]
    rcv = receivers[0]
    p = params

    pad = EP - E
    ef_p = jnp.concatenate([ef, jnp.zeros((pad, DE), jnp.float32)],
                           axis=0).astype(jnp.bfloat16).reshape(NH, EH, DE)
    snd_p = jnp.concatenate([snd, jnp.zeros((pad,), jnp.int32)])
    rcv_p = jnp.concatenate([rcv, jnp.full((pad,), NP_, jnp.int32)])
    snd_g = snd_p.reshape(NH, NT, NCH, CHUNK)
    rcv_g = rcv_p.reshape(NH, NT, NCH, CHUNK)
    rcv_s = rcv_p.reshape(NH, NS, SNCH, CHUNK)
    tau_arr = jnp.reshape(tau, (1, 1)).astype(jnp.float32)

    gather_sc, scatter_sc = _sc_kernels()

    BLK = 1024
    r1 = lambda a: a.reshape(1, -1)
    bf = lambda a: a.astype(jnp.bfloat16)
    edge_w = [bf(p['W_em1']), r1(p['b_em1']), bf(p['W_em2']), r1(p['b_em2']),
              p['em_cw'], r1(p['em_cb']), p['em_sw'], r1(p['em_sb']),
              p['em_ow'], r1(p['em_ob']),
              bf(p['W_pe1']), r1(p['b_pe1']), bf(p['W_pe2']), r1(p['b_pe2']),
              p['pe_cw'], r1(p['pe_cb']), p['pe_sw'], r1(p['pe_sb']),
              p['pe_ow'], r1(p['pe_ob'])]

    def edge_tc(ef_h, rs, ps):
        return pl.pallas_call(
            _edge_body,
            grid=(EH // BLK,),
            in_specs=[
                pl.BlockSpec((1, 1), lambda i: (0, 0),
                             memory_space=pltpu.SMEM),
                pl.BlockSpec((BLK, DE), lambda i: (i, 0)),
                pl.BlockSpec((BLK, D), lambda i: (i, 0)),
                pl.BlockSpec((BLK, D), lambda i: (i, 0)),
            ] + [_full(w.shape) for w in edge_w],
            out_specs=pl.BlockSpec((BLK, D), lambda i: (i, 0)),
            out_shape=jax.ShapeDtypeStruct((EH, D), jnp.float32),
        )(tau_arr, ef_h, rs, ps, *edge_w)

    zsum = jnp.zeros((CHUNK, D), jnp.float32)
    ones_c = jnp.ones((CHUNK, D), jnp.float32)

    rs0, ps0 = gather_sc(rn, pn, snd_g[0], rcv_g[0])
    rs1, ps1 = gather_sc(rn, pn, snd_g[1], rcv_g[1])
    e0 = edge_tc(ef_p[0], rs0, ps0)
    e1 = edge_tc(ef_p[1], rs1, ps1)
    psum0, pcnt0 = scatter_sc(e0, rcv_s[0], zsum, ones_c)
    psum1, pcnt1 = scatter_sc(e1, rcv_s[1], zsum, ones_c)

    BLKN = HALF // 2
    node_w = [p['W_pn1'], r1(p['b_pn1']), p['W_pn2'], r1(p['b_pn2']),
              p['pn_cw'], r1(p['pn_cb']), p['pn_sw'], r1(p['pn_sb']),
              p['pn_ow'], r1(p['pn_ob']),
              p['W_d1'], r1(p['b_d1']), p['W_d2'], r1(p['b_d2'])]
    half_spec = pl.BlockSpec((1, BLKN, D), lambda i: (i // 2, i % 2, 0))
    out = pl.pallas_call(
        _node_body,
        grid=(2 * NC,),
        in_specs=[
            pl.BlockSpec((1, 1), lambda i: (0, 0), memory_space=pltpu.SMEM),
            pl.BlockSpec((BLKN, D), lambda i: (i, 0)),
            half_spec, half_spec, half_spec, half_spec,
        ] + [_full(w.shape) for w in node_w],
        out_specs=pl.BlockSpec((BLKN, OUT), lambda i: (i, 0)),
        out_shape=jax.ShapeDtypeStruct((NP_, OUT), jnp.float32),
    )(tau_arr, pn, psum0, pcnt0, psum1, pcnt1, *node_w)

    return out.reshape(1, NP_, OUT)


# speedup vs baseline: 4.3626x; 1.0106x over previous
import functools

import jax
import jax.numpy as jnp
from jax import lax
from jax.experimental import pallas as pl
from jax.experimental.pallas import tpu as pltpu
from jax.experimental.pallas import tpu_sc as plsc

NR = 2500
NP_ = 10000
E = 160000
D = 128
DE = 16
OUT = 4
CH = 16

NC = 2
NS = 16
NT = NC * NS
CHUNK = 128
EP = 163840
NH = 4
EH = EP // NH
TILE_E = EH // NT
NCH = TILE_E // CHUNK
HALF = 5056
JUNK = 64
ACCR = HALF + JUNK
TILE_ES = EH // NS
SNCH = TILE_ES // CHUNK

def _gather_body(rn_hbm, pn_hbm, snd_hbm, rcv_hbm, rs_hbm, ps_hbm,
                 sidx, ridx, rb0, pb0, rb1, pb1, sg0, sg1, sw0, sw1):
    c = lax.axis_index("c")
    s = lax.axis_index("s")
    wid = s * NC + c
    base = wid * TILE_E
    pltpu.sync_copy(snd_hbm.at[wid], sidx)
    pltpu.sync_copy(rcv_hbm.at[wid], ridx)

    def wofs(j):
        return base + j * CHUNK

    pltpu.async_copy(rn_hbm.at[sidx.at[0]], rb0, sg0)
    pltpu.async_copy(pn_hbm.at[ridx.at[0]], pb0, sg0)

    def pair(k, _):
        j0 = 2 * k
        j1 = j0 + 1
        pltpu.make_async_copy(rn_hbm.at[sidx.at[j0]], rb0, sg0).wait()
        pltpu.make_async_copy(pn_hbm.at[ridx.at[j0]], pb0, sg0).wait()

        @pl.when(k > 0)
        def _():
            pltpu.make_async_copy(rb1, rs_hbm.at[pl.ds(wofs(j0 - 1), CHUNK)],
                                  sw1).wait()
            pltpu.make_async_copy(pb1, ps_hbm.at[pl.ds(wofs(j0 - 1), CHUNK)],
                                  sw1).wait()

        pltpu.async_copy(rn_hbm.at[sidx.at[j1]], rb1, sg1)
        pltpu.async_copy(pn_hbm.at[ridx.at[j1]], pb1, sg1)
        pltpu.async_copy(rb0, rs_hbm.at[pl.ds(wofs(j0), CHUNK)], sw0)
        pltpu.async_copy(pb0, ps_hbm.at[pl.ds(wofs(j0), CHUNK)], sw0)

        pltpu.make_async_copy(rn_hbm.at[sidx.at[j1]], rb1, sg1).wait()
        pltpu.make_async_copy(pn_hbm.at[ridx.at[j1]], pb1, sg1).wait()
        pltpu.make_async_copy(rb0, rs_hbm.at[pl.ds(wofs(j0), CHUNK)],
                              sw0).wait()
        pltpu.make_async_copy(pb0, ps_hbm.at[pl.ds(wofs(j0), CHUNK)],
                              sw0).wait()

        @pl.when(k < NCH // 2 - 1)
        def _():
            pltpu.async_copy(rn_hbm.at[sidx.at[j0 + 2]], rb0, sg0)
            pltpu.async_copy(pn_hbm.at[ridx.at[j0 + 2]], pb0, sg0)

        pltpu.async_copy(rb1, rs_hbm.at[pl.ds(wofs(j1), CHUNK)], sw1)
        pltpu.async_copy(pb1, ps_hbm.at[pl.ds(wofs(j1), CHUNK)], sw1)
        return 0

    lax.fori_loop(0, NCH // 2, pair, 0)
    last = NCH - 1
    pltpu.make_async_copy(rb1, rs_hbm.at[pl.ds(wofs(last), CHUNK)], sw1).wait()
    pltpu.make_async_copy(pb1, ps_hbm.at[pl.ds(wofs(last), CHUNK)], sw1).wait()


@functools.lru_cache(maxsize=None)
def _sc_kernels():
    mesh = plsc.VectorSubcoreMesh(
        core_axis_name="c", subcore_axis_name="s",
        num_cores=NC, num_subcores=NS)
    gather = pl.kernel(
        _gather_body,
        out_type=[
            jax.ShapeDtypeStruct((EH, D), jnp.float32),
            jax.ShapeDtypeStruct((EH, D), jnp.float32),
        ],
        mesh=mesh,
        scratch_types=[
            pltpu.VMEM((NCH, CHUNK), jnp.int32),
            pltpu.VMEM((NCH, CHUNK), jnp.int32),
            pltpu.VMEM((CHUNK, D), jnp.float32),
            pltpu.VMEM((CHUNK, D), jnp.float32),
            pltpu.VMEM((CHUNK, D), jnp.float32),
            pltpu.VMEM((CHUNK, D), jnp.float32),
            pltpu.SemaphoreType.DMA,
            pltpu.SemaphoreType.DMA,
            pltpu.SemaphoreType.DMA,
            pltpu.SemaphoreType.DMA,
        ],
    )
    scatter = pl.kernel(
        _scatter_body,
        out_type=[
            jax.ShapeDtypeStruct((NC, HALF, D), jnp.float32),
            jax.ShapeDtypeStruct((NC, HALF, D), jnp.float32),
        ],
        mesh=mesh,
        scratch_types=[
            pltpu.VMEM((SNCH, CHUNK), jnp.int32),
            pltpu.VMEM((CHUNK, D), jnp.float32),
            pltpu.VMEM((CHUNK, D), jnp.float32),
            pltpu.VMEM_SHARED((ACCR, D), jnp.float32),
            pltpu.VMEM_SHARED((ACCR, D), jnp.float32),
        ],
    )
    return gather, scatter


def _scatter_body(e_hbm, rcv_hbm, zsum_hbm, ones_hbm,
                  psum_hbm, pcnt_hbm, idx, ebuf, ones_v, acc, cacc):
    c = lax.axis_index("c")
    s = lax.axis_index("s")
    base = s * TILE_ES
    lo = c * HALF
    pltpu.sync_copy(zsum_hbm, ebuf)
    nblk = ACCR // CHUNK
    nmax = (nblk + NS - 1) // NS

    def zinit(k, _):
        blk = s + k * NS

        @pl.when(blk < nblk)
        def _():
            pltpu.sync_copy(ebuf, acc.at[pl.ds(blk * CHUNK, CHUNK)])
            pltpu.sync_copy(ebuf, cacc.at[pl.ds(blk * CHUNK, CHUNK)])

        return 0

    lax.fori_loop(0, nmax, zinit, 0)
    pltpu.sync_copy(ones_hbm, ones_v)
    pltpu.sync_copy(rcv_hbm.at[s], idx)
    iota = lax.iota(jnp.int32, 16)

    def locz(j, _):
        for k in range(CHUNK // 16):
            g = idx[j, pl.ds(k * 16, 16)]
            t = g - lo
            valid = (t >= 0) & (t < HALF)
            jv = HALF + iota + 16 * (k % (JUNK // 16))
            idx[j, pl.ds(k * 16, 16)] = jnp.where(valid, t, jv)
        return 0

    lax.fori_loop(0, SNCH, locz, 0)
    plsc.subcore_barrier()

    def body(j, _):
        pltpu.sync_copy(e_hbm.at[pl.ds(base + j * CHUNK, CHUNK)], ebuf)
        pltpu.sync_copy(ebuf, acc.at[idx.at[j]], add=True)
        pltpu.sync_copy(ones_v, cacc.at[idx.at[j]], add=True)
        return 0

    lax.fori_loop(0, SNCH, body, 0)
    plsc.subcore_barrier()
    wb = HALF // 64
    wmax = (wb + NS - 1) // NS

    def wback(k, _):
        blk = s + k * NS

        @pl.when(blk < wb)
        def _():
            r0 = blk * 64
            pltpu.sync_copy(acc.at[pl.ds(r0, 64)], ebuf.at[pl.ds(0, 64)])
            pltpu.sync_copy(ebuf.at[pl.ds(0, 64)],
                            psum_hbm.at[c, pl.ds(r0, 64)])
            pltpu.sync_copy(cacc.at[pl.ds(r0, 64)], ebuf.at[pl.ds(64, 64)])
            pltpu.sync_copy(ebuf.at[pl.ds(64, 64)],
                            pcnt_hbm.at[c, pl.ds(r0, 64)])

        return 0

    lax.fori_loop(0, wmax, wback, 0)


def _swish(x):
    return x * jax.nn.sigmoid(x)


def _cn_vectors(tau, cw, cb, sw, sb, ow, ob):
    h = _swish(tau * cw + cb)
    scale = jnp.dot(h, sw, preferred_element_type=jnp.float32) + sb
    shift = jnp.dot(h, ow, preferred_element_type=jnp.float32) + ob
    return scale, shift


def _layernorm(x):
    m = jnp.mean(x, axis=-1, keepdims=True)
    xc = x - m
    v = jnp.mean(xc * xc, axis=-1, keepdims=True)
    return xc * lax.rsqrt(v + 1e-5)


def _edge_body(tau_ref, ef_ref, rs_ref, ps_ref,
               W_em1, b_em1, W_em2, b_em2,
               em_cw, em_cb, em_sw, em_sb, em_ow, em_ob,
               W_pe1, b_pe1, W_pe2, b_pe2,
               pe_cw, pe_cb, pe_sw, pe_sb, pe_ow, pe_ob,
               e_ref):
    tau = tau_ref[0, 0]
    sc_em, sh_em = _cn_vectors(tau, em_cw[...], em_cb[...], em_sw[...],
                               em_sb[...], em_ow[...], em_ob[...])
    sc_pe, sh_pe = _cn_vectors(tau, pe_cw[...], pe_cb[...], pe_sw[...],
                               pe_sb[...], pe_ow[...], pe_ob[...])
    x = ef_ref[...]
    h = _swish(jnp.dot(x, W_em1[...], preferred_element_type=jnp.float32)
               + b_em1[...])
    e0 = jnp.dot(h.astype(jnp.bfloat16), W_em2[...],
                 preferred_element_type=jnp.float32) + b_em2[...]
    e0 = _layernorm(e0) * (1.0 + sc_em) + sh_em
    m = (jnp.dot(e0.astype(jnp.bfloat16), W_pe1[...][:D],
                 preferred_element_type=jnp.float32)
         + jnp.dot(rs_ref[...], W_pe1[...][D:2 * D],
                   preferred_element_type=jnp.float32)
         + jnp.dot(ps_ref[...], W_pe1[...][2 * D:],
                   preferred_element_type=jnp.float32)
         + b_pe1[...])
    m = jnp.dot(_swish(m).astype(jnp.bfloat16), W_pe2[...],
                preferred_element_type=jnp.float32) \
        + b_pe2[...]
    m = _layernorm(m) * (1.0 + sc_pe) + sh_pe
    e_ref[...] = e0 + m


def _node_body(tau_ref, pn_ref, *rest):
    (psum0_ref, pcnt0_ref, psum1_ref, pcnt1_ref,
     psum2_ref, pcnt2_ref, psum3_ref, pcnt3_ref,
     W_pn1, b_pn1, W_pn2, b_pn2,
     pn_cw, pn_cb, pn_sw, pn_sb, pn_ow, pn_ob,
     W_d1, b_d1, W_d2, b_d2,
     out_ref) = rest
    tau = tau_ref[0, 0]
    sc_pn, sh_pn = _cn_vectors(tau, pn_cw[...], pn_cb[...], pn_sw[...],
                               pn_sb[...], pn_ow[...], pn_ob[...])
    pn = pn_ref[...]
    s = (psum0_ref[0] + psum1_ref[0]) + (psum2_ref[0] + psum3_ref[0])
    cnt = (pcnt0_ref[0, :, 0:1] + pcnt1_ref[0, :, 0:1]) \
        + (pcnt2_ref[0, :, 0:1] + pcnt3_ref[0, :, 0:1])
    agg = s / jnp.maximum(cnt, 1.0)
    h = _swish(jnp.dot(pn, W_pn1[...][:D], preferred_element_type=jnp.float32)
               + jnp.dot(agg, W_pn1[...][D:],
                         preferred_element_type=jnp.float32)
               + b_pn1[...])
    h = jnp.dot(h, W_pn2[...], preferred_element_type=jnp.float32) + b_pn2[...]
    h = _layernorm(h) * (1.0 + sc_pn) + sh_pn
    pn = pn + h
    d = _swish(jnp.dot(pn, W_d1[...], preferred_element_type=jnp.float32)
               + b_d1[...])
    out_ref[...] = jnp.dot(d, W_d2[...], preferred_element_type=jnp.float32) \
        + b_d2[...]


def _full(shape):
    nd = len(shape)
    return pl.BlockSpec(shape, lambda i: (0,) * nd)


def kernel(rnode_features, pnode_features, edge_features, senders, receivers,
           tau, params):
    rn = rnode_features[0]
    pn = pnode_features[0]
    ef = edge_features[0]
    snd = senders[0]
    rcv = receivers[0]
    p = params

    pad = EP - E
    ef_p = jnp.concatenate([ef, jnp.zeros((pad, DE), jnp.float32)],
                           axis=0).astype(jnp.bfloat16).reshape(NH, EH, DE)
    snd_p = jnp.concatenate([snd, jnp.zeros((pad,), jnp.int32)])
    rcv_p = jnp.concatenate([rcv, jnp.full((pad,), NP_, jnp.int32)])
    snd_g = snd_p.reshape(NH, NT, NCH, CHUNK)
    rcv_g = rcv_p.reshape(NH, NT, NCH, CHUNK)
    rcv_s = rcv_p.reshape(NH, NS, SNCH, CHUNK)
    tau_arr = jnp.reshape(tau, (1, 1)).astype(jnp.float32)

    gather_sc, scatter_sc = _sc_kernels()

    BLK = 1024
    r1 = lambda a: a.reshape(1, -1)
    bf = lambda a: a.astype(jnp.bfloat16)
    edge_w = [bf(p['W_em1']), r1(p['b_em1']), bf(p['W_em2']), r1(p['b_em2']),
              p['em_cw'], r1(p['em_cb']), p['em_sw'], r1(p['em_sb']),
              p['em_ow'], r1(p['em_ob']),
              bf(p['W_pe1']), r1(p['b_pe1']), bf(p['W_pe2']), r1(p['b_pe2']),
              p['pe_cw'], r1(p['pe_cb']), p['pe_sw'], r1(p['pe_sb']),
              p['pe_ow'], r1(p['pe_ob'])]

    def edge_tc(ef_h, rs, ps):
        return pl.pallas_call(
            _edge_body,
            grid=(EH // BLK,),
            in_specs=[
                pl.BlockSpec((1, 1), lambda i: (0, 0),
                             memory_space=pltpu.SMEM),
                pl.BlockSpec((BLK, DE), lambda i: (i, 0)),
                pl.BlockSpec((BLK, D), lambda i: (i, 0)),
                pl.BlockSpec((BLK, D), lambda i: (i, 0)),
            ] + [_full(w.shape) for w in edge_w],
            out_specs=pl.BlockSpec((BLK, D), lambda i: (i, 0)),
            out_shape=jax.ShapeDtypeStruct((EH, D), jnp.float32),
        )(tau_arr, ef_h, rs, ps, *edge_w)

    zsum = jnp.zeros((CHUNK, D), jnp.float32)
    ones_c = jnp.ones((CHUNK, D), jnp.float32)

    gp = [gather_sc(rn, pn, snd_g[h], rcv_g[h]) for h in range(NH)]
    es = [edge_tc(ef_p[h], rs, ps) for h, (rs, ps) in enumerate(gp)]
    parts = [scatter_sc(es[h], rcv_s[h], zsum, ones_c) for h in range(NH)]

    BLKN = HALF // 2
    node_w = [p['W_pn1'], r1(p['b_pn1']), p['W_pn2'], r1(p['b_pn2']),
              p['pn_cw'], r1(p['pn_cb']), p['pn_sw'], r1(p['pn_sb']),
              p['pn_ow'], r1(p['pn_ob']),
              p['W_d1'], r1(p['b_d1']), p['W_d2'], r1(p['b_d2'])]
    half_spec = pl.BlockSpec((1, BLKN, D), lambda i: (i // 2, i % 2, 0))
    out = pl.pallas_call(
        _node_body,
        grid=(2 * NC,),
        in_specs=[
            pl.BlockSpec((1, 1), lambda i: (0, 0), memory_space=pltpu.SMEM),
            pl.BlockSpec((BLKN, D), lambda i: (i, 0)),
        ] + [half_spec] * (2 * NH) + [_full(w.shape) for w in node_w],
        out_specs=pl.BlockSpec((BLKN, OUT), lambda i: (i, 0)),
        out_shape=jax.ShapeDtypeStruct((NP_, OUT), jnp.float32),
    )(tau_arr, pn, *[a for pr in parts for a in pr], *node_w)

    return out.reshape(1, NP_, OUT)
